# baseline (speedup 1.0000x reference)
import functools

import jax
import jax.numpy as jnp
from jax import lax
from jax.experimental import pallas as pl
from jax.experimental.pallas import tpu as pltpu
from jax.experimental.pallas import tpu_sc as plsc

def _dot_bf16(a, w):
    return jnp.dot(a.astype(jnp.bfloat16), w.astype(jnp.bfloat16),
                   preferred_element_type=jnp.float32)


_N = 50000
_E = 800000
_H = 64
_NPAD = 50176
_NBLK = 16
_BLK = _NPAD // _NBLK


def _mlp_block(eps_ref, h_ref, agg_ref, w1_ref, b1_ref, w2_ref, b2_ref,
               z2_ref, s1_ref, *, first_layer: bool):
    i = pl.program_id(0)
    if first_layer:
        z0 = (1.0 + eps_ref[0, 0]) * h_ref[:, 0:1] + agg_ref[:, 0:1]
        a = z0 * w1_ref[0:1, :] + b1_ref[...]
    else:
        z = (1.0 + eps_ref[0, 0]) * h_ref[...] + agg_ref[...]
        a = _dot_bf16(z, w1_ref[...])
        a = a + b1_ref[...]
    a = jnp.maximum(a, 0.0)
    z2 = _dot_bf16(a, w2_ref[...])
    z2 = jnp.maximum(z2 + b2_ref[...], 0.0)
    z2_ref[...] = z2
    rows = lax.broadcasted_iota(jnp.int32, (_BLK, 1), 0) + i * _BLK
    zm = jnp.where(rows < _N, z2, 0.0)
    s1 = jnp.sum(zm, axis=0).reshape(1, _H)

    @pl.when(i == 0)
    def _():
        s1_ref[...] = s1

    @pl.when(i > 0)
    def _():
        s1_ref[...] += s1


def _tc_mlp(h, agg, eps, w1, b1, w2, b2, first_layer):
    din = h.shape[1]
    return pl.pallas_call(
        functools.partial(_mlp_block, first_layer=first_layer),
        grid=(_NBLK,),
        in_specs=[
            pl.BlockSpec((1, 1), lambda i: (0, 0)),
            pl.BlockSpec((_BLK, din), lambda i: (i, 0)),
            pl.BlockSpec((_BLK, agg.shape[1]), lambda i: (i, 0)),
            pl.BlockSpec(w1.shape, lambda i: (0, 0)),
            pl.BlockSpec((1, _H), lambda i: (0, 0)),
            pl.BlockSpec((_H, _H), lambda i: (0, 0)),
            pl.BlockSpec((1, _H), lambda i: (0, 0)),
        ],
        out_specs=[
            pl.BlockSpec((_BLK, _H), lambda i: (i, 0)),
            pl.BlockSpec((1, _H), lambda i: (0, 0)),
        ],
        out_shape=[
            jax.ShapeDtypeStruct((_NPAD, _H), jnp.float32),
            jax.ShapeDtypeStruct((1, _H), jnp.float32),
        ],
    )(eps.reshape(1, 1), h, agg, w1, b1.reshape(1, _H), w2, b2.reshape(1, _H))


def _norm2_block(z2_ref, s1_ref, g_ref, be_ref, h_ref, s2_ref):
    t = pl.program_id(0)
    i = pl.program_id(1)
    mu = s1_ref[...] / _N

    @pl.when(t == 0)
    def _():
        rows = lax.broadcasted_iota(jnp.int32, (_BLK, 1), 0) + i * _BLK
        dlt = jnp.where(rows < _N, z2_ref[...] - mu, 0.0)
        ssq = jnp.sum(dlt * dlt, axis=0).reshape(1, _H)

        @pl.when(i == 0)
        def _():
            s2_ref[...] = ssq

        @pl.when(i > 0)
        def _():
            s2_ref[...] += ssq

    @pl.when(t == 1)
    def _():
        var = s2_ref[...] / _N
        h_ref[...] = (g_ref[...] * (z2_ref[...] - mu) / jnp.sqrt(var + 1e-5)
                      + be_ref[...])


def _tc_norm(z2, s1, g, be):
    return pl.pallas_call(
        _norm2_block,
        grid=(2, _NBLK),
        in_specs=[
            pl.BlockSpec((_BLK, _H), lambda t, i: (i, 0)),
            pl.BlockSpec((1, _H), lambda t, i: (0, 0)),
            pl.BlockSpec((1, _H), lambda t, i: (0, 0)),
            pl.BlockSpec((1, _H), lambda t, i: (0, 0)),
        ],
        out_specs=pl.BlockSpec((_BLK, _H), lambda t, i: (i, 0)),
        out_shape=jax.ShapeDtypeStruct((_NPAD, _H), jnp.float32),
        scratch_shapes=[pltpu.VMEM((1, _H), jnp.float32)],
    )(z2, s1, g.reshape(1, _H), be.reshape(1, _H))


def _norm_fc_block(z2_ref, s1_ref, g_ref, be_ref, fcw_ref, fcb_ref, o_ref,
                   s2_ref):
    t = pl.program_id(0)
    i = pl.program_id(1)
    mu = s1_ref[...] / _N

    @pl.when(t == 0)
    def _():
        rows = lax.broadcasted_iota(jnp.int32, (_BLK, 1), 0) + i * _BLK
        dlt = jnp.where(rows < _N, z2_ref[...] - mu, 0.0)
        ssq = jnp.sum(dlt * dlt, axis=0).reshape(1, _H)

        @pl.when(i == 0)
        def _():
            s2_ref[...] = ssq

        @pl.when(i > 0)
        def _():
            s2_ref[...] += ssq

    @pl.when(t == 1)
    def _():
        var = s2_ref[...] / _N
        hn = (g_ref[...] * (z2_ref[...] - mu) / jnp.sqrt(var + 1e-5)
              + be_ref[...])
        o_ref[...] = _dot_bf16(hn, fcw_ref[...]) + fcb_ref[0, 0]


def _tc_norm_fc(z2, s1, g, be, fcw, fcb):
    return pl.pallas_call(
        _norm_fc_block,
        grid=(2, _NBLK),
        in_specs=[
            pl.BlockSpec((_BLK, _H), lambda t, i: (i, 0)),
            pl.BlockSpec((1, _H), lambda t, i: (0, 0)),
            pl.BlockSpec((1, _H), lambda t, i: (0, 0)),
            pl.BlockSpec((1, _H), lambda t, i: (0, 0)),
            pl.BlockSpec((_H, 1), lambda t, i: (0, 0)),
            pl.BlockSpec((1, 1), lambda t, i: (0, 0)),
        ],
        out_specs=pl.BlockSpec((_BLK, 1), lambda t, i: (i, 0)),
        out_shape=jax.ShapeDtypeStruct((_NPAD, 1), jnp.float32),
        scratch_shapes=[pltpu.VMEM((1, _H), jnp.float32)],
    )(z2, s1, g.reshape(1, _H), be.reshape(1, _H), fcw, fcb.reshape(1, 1))


_HALF = _NPAD // 2
_NDUMP = 256
_ACC = _HALF + _NDUMP
_EPAD = 819200
_EPT = _EPAD // 16
_EBLK = 2048
_NEBLK = _EPT // _EBLK
_ZR = 98
_RPS = _HALF // 16


def _make_sc_agg(d):
    mesh = plsc.VectorSubcoreMesh(core_axis_name="c", subcore_axis_name="s")

    @functools.partial(
        pl.kernel,
        out_type=jax.ShapeDtypeStruct((_NPAD, d), jnp.float32),
        mesh=mesh,
        compiler_params=pltpu.CompilerParams(use_tc_tiling_on_sc=False),
        scratch_types=[
            pltpu.VMEM((_EBLK,), jnp.int32),
            pltpu.VMEM((_EBLK,), jnp.int32),
            pltpu.VMEM((16, 128), jnp.int32),
            pltpu.VMEM((128, d), jnp.float32),
            pltpu.VMEM((128, d), jnp.float32),
            pltpu.VMEM((_ZR, d), jnp.float32),
            pltpu.VMEM_SHARED((_ACC, d), jnp.float32),
            pltpu.SemaphoreType.DMA,
            pltpu.SemaphoreType.DMA,
        ],
    )
    def agg_kernel(h_hbm, src_hbm, dst_hbm, out_hbm, src_v, dst_v, ldst_v,
                   rows_a, rows_b, zbuf, acc, sem_a, sem_b):
        cid = lax.axis_index("c")
        sid = lax.axis_index("s")
        lo = cid * _HALF
        zvec = jnp.zeros((16,), jnp.float32)

        @pl.loop(0, _ZR)
        def _(r):
            for c in range(0, d, 16):
                zbuf[r, pl.ds(c, 16)] = zvec

        for j in range(16):
            pltpu.sync_copy(zbuf, acc.at[pl.ds(sid * _RPS + j * _ZR, _ZR)])
        plsc.subcore_barrier()

        base = sid * _EPT

        @pl.loop(0, _NEBLK)
        def _(b):
            off = base + b * _EBLK
            pltpu.sync_copy(src_hbm.at[pl.ds(off, _EBLK)], src_v)
            pltpu.sync_copy(dst_hbm.at[pl.ds(off, _EBLK)], dst_v)

            @pl.loop(0, _EBLK // 16)
            def _(i):
                dd = dst_v[pl.ds(i * 16, 16)]
                m = (dd >= lo) & (dd < lo + _HALF)
                ld = jnp.where(m, dd - lo, _HALF + (dd & (_NDUMP - 1)))
                ldst_v[i // 8, pl.ds((i % 8) * 16, 16)] = ld

            bufs = (rows_a, rows_b)
            sems = (sem_a, sem_b)
            cps = [None, None]
            cps[0] = pltpu.async_copy(
                h_hbm.at[src_v.at[pl.ds(0, 128)]], rows_a, sem_a)
            for j in range(16):
                cur = j & 1
                if j < 15:
                    cps[1 - cur] = pltpu.async_copy(
                        h_hbm.at[src_v.at[pl.ds((j + 1) * 128, 128)]],
                        bufs[1 - cur], sems[1 - cur])
                cps[cur].wait()
                pltpu.sync_copy(bufs[cur], acc.at[ldst_v.at[j]], add=True)

        plsc.subcore_barrier()
        pltpu.sync_copy(
            acc.at[pl.ds(sid * _RPS, _RPS)],
            out_hbm.at[pl.ds(cid * _HALF + sid * _RPS, _RPS)])

    return agg_kernel


_sc_agg_cache = {}


def _sc_agg(d, h, src_p, dst_p):
    if d not in _sc_agg_cache:
        _sc_agg_cache[d] = _make_sc_agg(d)
    return _sc_agg_cache[d](h, src_p, dst_p)


def kernel(x, edge_index,
           eps0, W1_0, b1_0, W2_0, b2_0, g0, be0,
           eps1, W1_1, b1_1, W2_1, b2_1, g1, be1,
           eps2, W1_2, b1_2, W2_2, b2_2, g2, be2,
           eps3, W1_3, b1_3, W2_3, b2_3, g3, be3,
           fcW, fcb):
    pad_iota = lax.iota(jnp.int32, _EPAD - _E)
    src_p = jnp.concatenate([edge_index[0], pad_iota])
    dst_p = jnp.concatenate([edge_index[1], (1 << 20) + pad_iota])
    x16 = jnp.pad(x, ((0, _NPAD - _N), (0, 15)))

    agg0 = _sc_agg(16, x16, src_p, dst_p)
    z2, s1 = _tc_mlp(x16, agg0, eps0, W1_0, b1_0, W2_0, b2_0, True)
    h = _tc_norm(z2, s1, g0, be0)

    for (eps, w1, b1, w2, b2, g, be) in (
            (eps1, W1_1, b1_1, W2_1, b2_1, g1, be1),
            (eps2, W1_2, b1_2, W2_2, b2_2, g2, be2)):
        agg = _sc_agg(_H, h, src_p, dst_p)
        z2, s1 = _tc_mlp(h, agg, eps, w1, b1, w2, b2, False)
        h = _tc_norm(z2, s1, g, be)

    agg = _sc_agg(_H, h, src_p, dst_p)
    z2, s1 = _tc_mlp(h, agg, eps3, W1_3, b1_3, W2_3, b2_3, False)
    out = _tc_norm_fc(z2, s1, g3, be3, fcW, fcb)
    return out[:_N]


# speedup vs baseline: 1.0096x; 1.0096x over previous
import functools

import jax
import jax.numpy as jnp
from jax import lax
from jax.experimental import pallas as pl
from jax.experimental.pallas import tpu as pltpu
from jax.experimental.pallas import tpu_sc as plsc

def _dot_bf16(a, w):
    return jnp.dot(a.astype(jnp.bfloat16), w.astype(jnp.bfloat16),
                   preferred_element_type=jnp.float32)


_N = 50000
_E = 800000
_H = 64
_NPAD = 50176
_NBLK = 16
_BLK = _NPAD // _NBLK


def _mlp_block(eps_ref, h_ref, agg_ref, w1_ref, b1_ref, w2_ref, b2_ref,
               z2_ref, s1_ref, *, first_layer: bool):
    i = pl.program_id(0)
    if first_layer:
        z0 = (1.0 + eps_ref[0, 0]) * h_ref[:, 0:1] + agg_ref[:, 0:1]
        a = z0 * w1_ref[0:1, :] + b1_ref[...]
    else:
        z = (1.0 + eps_ref[0, 0]) * h_ref[...] + agg_ref[...]
        a = _dot_bf16(z, w1_ref[...])
        a = a + b1_ref[...]
    a = jnp.maximum(a, 0.0)
    z2 = _dot_bf16(a, w2_ref[...])
    z2 = jnp.maximum(z2 + b2_ref[...], 0.0)
    z2_ref[...] = z2
    rows = lax.broadcasted_iota(jnp.int32, (_BLK, 1), 0) + i * _BLK
    zm = jnp.where(rows < _N, z2, 0.0)
    s1 = jnp.sum(zm, axis=0).reshape(1, _H)

    @pl.when(i == 0)
    def _():
        s1_ref[...] = s1

    @pl.when(i > 0)
    def _():
        s1_ref[...] += s1


def _tc_mlp(h, agg, eps, w1, b1, w2, b2, first_layer):
    din = h.shape[1]
    return pl.pallas_call(
        functools.partial(_mlp_block, first_layer=first_layer),
        grid=(_NBLK,),
        in_specs=[
            pl.BlockSpec((1, 1), lambda i: (0, 0)),
            pl.BlockSpec((_BLK, din), lambda i: (i, 0)),
            pl.BlockSpec((_BLK, agg.shape[1]), lambda i: (i, 0)),
            pl.BlockSpec(w1.shape, lambda i: (0, 0)),
            pl.BlockSpec((1, _H), lambda i: (0, 0)),
            pl.BlockSpec((_H, _H), lambda i: (0, 0)),
            pl.BlockSpec((1, _H), lambda i: (0, 0)),
        ],
        out_specs=[
            pl.BlockSpec((_BLK, _H), lambda i: (i, 0)),
            pl.BlockSpec((1, _H), lambda i: (0, 0)),
        ],
        out_shape=[
            jax.ShapeDtypeStruct((_NPAD, _H), jnp.float32),
            jax.ShapeDtypeStruct((1, _H), jnp.float32),
        ],
    )(eps.reshape(1, 1), h, agg, w1, b1.reshape(1, _H), w2, b2.reshape(1, _H))


def _var_block(z2_ref, s1_ref, sc_ref):
    i = pl.program_id(0)
    mu = s1_ref[...] / _N
    rows = lax.broadcasted_iota(jnp.int32, (_BLK, 1), 0) + i * _BLK
    d = jnp.where(rows < _N, z2_ref[...] - mu, 0.0)
    s = jnp.sum(d * d, axis=0).reshape(1, _H)

    @pl.when(i == 0)
    def _():
        sc_ref[...] = s

    @pl.when(i > 0)
    def _():
        sc_ref[...] += s


def _tc_var(z2, s1):
    return pl.pallas_call(
        _var_block,
        grid=(_NBLK,),
        in_specs=[
            pl.BlockSpec((_BLK, _H), lambda i: (i, 0)),
            pl.BlockSpec((1, _H), lambda i: (0, 0)),
        ],
        out_specs=pl.BlockSpec((1, _H), lambda i: (0, 0)),
        out_shape=jax.ShapeDtypeStruct((1, _H), jnp.float32),
    )(z2, s1)


def _norm_block(z2_ref, s1_ref, s2_ref, g_ref, be_ref, h_ref):
    mu = s1_ref[...] / _N
    var = s2_ref[...] / _N
    h_ref[...] = (g_ref[...] * (z2_ref[...] - mu) / jnp.sqrt(var + 1e-5)
                  + be_ref[...])


def _tc_norm(z2, s1, s2, g, be):
    return pl.pallas_call(
        _norm_block,
        grid=(_NBLK,),
        in_specs=[
            pl.BlockSpec((_BLK, _H), lambda i: (i, 0)),
            pl.BlockSpec((1, _H), lambda i: (0, 0)),
            pl.BlockSpec((1, _H), lambda i: (0, 0)),
            pl.BlockSpec((1, _H), lambda i: (0, 0)),
            pl.BlockSpec((1, _H), lambda i: (0, 0)),
        ],
        out_specs=pl.BlockSpec((_BLK, _H), lambda i: (i, 0)),
        out_shape=jax.ShapeDtypeStruct((_NPAD, _H), jnp.float32),
    )(z2, s1, s2, g.reshape(1, _H), be.reshape(1, _H))


def _norm_fc_block(z2_ref, s1_ref, s2_ref, g_ref, be_ref, fcw_ref, fcb_ref,
                   o_ref):
    mu = s1_ref[...] / _N
    var = s2_ref[...] / _N
    hn = g_ref[...] * (z2_ref[...] - mu) / jnp.sqrt(var + 1e-5) + be_ref[...]
    o_ref[...] = _dot_bf16(hn, fcw_ref[...]) + fcb_ref[0, 0]


def _tc_norm_fc(z2, s1, s2, g, be, fcw, fcb):
    return pl.pallas_call(
        _norm_fc_block,
        grid=(_NBLK,),
        in_specs=[
            pl.BlockSpec((_BLK, _H), lambda i: (i, 0)),
            pl.BlockSpec((1, _H), lambda i: (0, 0)),
            pl.BlockSpec((1, _H), lambda i: (0, 0)),
            pl.BlockSpec((1, _H), lambda i: (0, 0)),
            pl.BlockSpec((1, _H), lambda i: (0, 0)),
            pl.BlockSpec((_H, 1), lambda i: (0, 0)),
            pl.BlockSpec((1, 1), lambda i: (0, 0)),
        ],
        out_specs=pl.BlockSpec((_BLK, 1), lambda i: (i, 0)),
        out_shape=jax.ShapeDtypeStruct((_NPAD, 1), jnp.float32),
    )(z2, s1, s2, g.reshape(1, _H), be.reshape(1, _H), fcw,
      fcb.reshape(1, 1))


_HALF = _NPAD // 2
_NDUMP = 256
_ACC = _HALF + _NDUMP
_EPAD = 819200
_EPT = _EPAD // 16
_EBLK = 2048
_NEBLK = _EPT // _EBLK
_ZR = 98
_RPS = _HALF // 16


def _make_sc_agg(d):
    mesh = plsc.VectorSubcoreMesh(core_axis_name="c", subcore_axis_name="s")

    @functools.partial(
        pl.kernel,
        out_type=jax.ShapeDtypeStruct((_NPAD, d), jnp.float32),
        mesh=mesh,
        compiler_params=pltpu.CompilerParams(use_tc_tiling_on_sc=False),
        scratch_types=[
            pltpu.VMEM((_EBLK,), jnp.int32),
            pltpu.VMEM((_EBLK,), jnp.int32),
            pltpu.VMEM((16, 128), jnp.int32),
            pltpu.VMEM((128, d), jnp.float32),
            pltpu.VMEM((128, d), jnp.float32),
            pltpu.VMEM((_ZR, d), jnp.float32),
            pltpu.VMEM_SHARED((_ACC, d), jnp.float32),
            pltpu.SemaphoreType.DMA,
            pltpu.SemaphoreType.DMA,
        ],
    )
    def agg_kernel(h_hbm, src_hbm, dst_hbm, out_hbm, src_v, dst_v, ldst_v,
                   rows_a, rows_b, zbuf, acc, sem_a, sem_b):
        cid = lax.axis_index("c")
        sid = lax.axis_index("s")
        lo = cid * _HALF
        zvec = jnp.zeros((16,), jnp.float32)

        @pl.loop(0, _ZR)
        def _(r):
            for c in range(0, d, 16):
                zbuf[r, pl.ds(c, 16)] = zvec

        for j in range(16):
            pltpu.sync_copy(zbuf, acc.at[pl.ds(sid * _RPS + j * _ZR, _ZR)])
        plsc.subcore_barrier()

        base = sid * _EPT

        @pl.loop(0, _NEBLK)
        def _(b):
            off = base + b * _EBLK
            pltpu.sync_copy(src_hbm.at[pl.ds(off, _EBLK)], src_v)
            pltpu.sync_copy(dst_hbm.at[pl.ds(off, _EBLK)], dst_v)

            @pl.loop(0, _EBLK // 16)
            def _(i):
                dd = dst_v[pl.ds(i * 16, 16)]
                m = (dd >= lo) & (dd < lo + _HALF)
                ld = jnp.where(m, dd - lo, _HALF + (dd & (_NDUMP - 1)))
                ldst_v[i // 8, pl.ds((i % 8) * 16, 16)] = ld

            bufs = (rows_a, rows_b)
            sems = (sem_a, sem_b)
            cps = [None, None]
            cps[0] = pltpu.async_copy(
                h_hbm.at[src_v.at[pl.ds(0, 128)]], rows_a, sem_a)
            for j in range(16):
                cur = j & 1
                if j < 15:
                    cps[1 - cur] = pltpu.async_copy(
                        h_hbm.at[src_v.at[pl.ds((j + 1) * 128, 128)]],
                        bufs[1 - cur], sems[1 - cur])
                cps[cur].wait()
                pltpu.sync_copy(bufs[cur], acc.at[ldst_v.at[j]], add=True)

        plsc.subcore_barrier()
        pltpu.sync_copy(
            acc.at[pl.ds(sid * _RPS, _RPS)],
            out_hbm.at[pl.ds(cid * _HALF + sid * _RPS, _RPS)])

    return agg_kernel


_sc_agg_cache = {}


def _sc_agg(d, h, src_p, dst_p):
    if d not in _sc_agg_cache:
        _sc_agg_cache[d] = _make_sc_agg(d)
    return _sc_agg_cache[d](h, src_p, dst_p)


def kernel(x, edge_index,
           eps0, W1_0, b1_0, W2_0, b2_0, g0, be0,
           eps1, W1_1, b1_1, W2_1, b2_1, g1, be1,
           eps2, W1_2, b1_2, W2_2, b2_2, g2, be2,
           eps3, W1_3, b1_3, W2_3, b2_3, g3, be3,
           fcW, fcb):
    pad_iota = lax.iota(jnp.int32, _EPAD - _E)
    src_p = jnp.concatenate([edge_index[0], pad_iota])
    dst_p = jnp.concatenate([edge_index[1], (1 << 20) + pad_iota])
    x16 = jnp.pad(x, ((0, _NPAD - _N), (0, 15)))

    agg0 = _sc_agg(16, x16, src_p, dst_p)
    z2, s1 = _tc_mlp(x16, agg0, eps0, W1_0, b1_0, W2_0, b2_0, True)
    h = _tc_norm(z2, s1, _tc_var(z2, s1), g0, be0)

    for (eps, w1, b1, w2, b2, g, be) in (
            (eps1, W1_1, b1_1, W2_1, b2_1, g1, be1),
            (eps2, W1_2, b1_2, W2_2, b2_2, g2, be2)):
        agg = _sc_agg(_H, h, src_p, dst_p)
        z2, s1 = _tc_mlp(h, agg, eps, w1, b1, w2, b2, False)
        h = _tc_norm(z2, s1, _tc_var(z2, s1), g, be)

    agg = _sc_agg(_H, h, src_p, dst_p)
    z2, s1 = _tc_mlp(h, agg, eps3, W1_3, b1_3, W2_3, b2_3, False)
    out = _tc_norm_fc(z2, s1, _tc_var(z2, s1), g3, be3, fcW, fcb)
    return out[:_N]
